# Initial kernel scaffold; baseline (speedup 1.0000x reference)
import jax
import jax.numpy as jnp
from jax.experimental import pallas as pl


def kernel(coords, pe):
    raise NotImplementedError("write your pallas kernel here")


# speedup vs baseline: 1.1326x; 1.1326x over previous
import functools

import jax
import jax.numpy as jnp
from jax import lax
from jax.experimental import pallas as pl
from jax.experimental.pallas import tpu as pltpu
from jax.experimental.pallas import tpu_sc as plsc

D_MODEL = 256
MAX_LEN = 100
B_TOTAL = 16 * 128 * 50
GRID = 100
SUB = 8
LANE = 128
NC = 2
NS = 16
NW = NC * NS
BPW = B_TOTAL // NW
CH = 128
NCH = BPW // CH
NPAIR = NCH // 2


def _prep_body(xs_ref, ys_ref, pe_row_ref, pe_full_ref, idx_ref, p2_ref):
    qx = (((xs_ref[0] + 50.0) / 100.0) * (MAX_LEN - 1)).astype(jnp.int32)
    qy = (((ys_ref[0] + 50.0) / 100.0) * (MAX_LEN - 1)).astype(jnp.int32)
    qx = jnp.clip(qx, 0, MAX_LEN - 1)
    qy = jnp.clip(qy, 0, MAX_LEN - 1)
    idx_ref[0] = qx * MAX_LEN + qy
    p2_ref[0] = pe_row_ref[0] + pe_full_ref[...]


def _prep(xs, ys, pe):
    return pl.pallas_call(
        _prep_body,
        grid=(GRID,),
        in_specs=[
            pl.BlockSpec((1, SUB, LANE), lambda i: (i, 0, 0)),
            pl.BlockSpec((1, SUB, LANE), lambda i: (i, 0, 0)),
            pl.BlockSpec((1, 1, D_MODEL), lambda i: (i, 0, 0)),
            pl.BlockSpec((MAX_LEN, D_MODEL), lambda i: (0, 0)),
        ],
        out_specs=[
            pl.BlockSpec((1, SUB, LANE), lambda i: (i, 0, 0)),
            pl.BlockSpec((1, MAX_LEN, D_MODEL), lambda i: (i, 0, 0)),
        ],
        out_shape=[
            jax.ShapeDtypeStruct((GRID, SUB, LANE), jnp.int32),
            jax.ShapeDtypeStruct((MAX_LEN, MAX_LEN, D_MODEL), jnp.float32),
        ],
    )(xs, ys, pe.reshape(MAX_LEN, 1, D_MODEL), pe)


@functools.partial(
    pl.kernel,
    mesh=plsc.VectorSubcoreMesh(core_axis_name="c", subcore_axis_name="s"),
    out_type=jax.ShapeDtypeStruct((B_TOTAL, D_MODEL), jnp.float32),
    scratch_types=[
        pltpu.VMEM((NCH, CH), jnp.int32),
        pltpu.VMEM((CH, D_MODEL), jnp.float32),
        pltpu.VMEM((CH, D_MODEL), jnp.float32),
        pltpu.SemaphoreType.DMA,
        pltpu.SemaphoreType.DMA,
    ],
)
def _sc_gather(p2_hbm, idx_hbm, out_hbm, idx_v, buf0, buf1, sem0, sem1):
    wid = lax.axis_index("s") * NC + lax.axis_index("c")
    base = wid * BPW
    pltpu.sync_copy(idx_hbm.at[wid], idx_v)

    pltpu.async_copy(p2_hbm.at[idx_v.at[0]], buf0, sem0).wait()

    def body(g, carry):
        c0 = 2 * g
        cp1 = pltpu.async_copy(p2_hbm.at[idx_v.at[c0 + 1]], buf1, sem1)
        pltpu.sync_copy(buf0, out_hbm.at[pl.ds(base + c0 * CH, CH)])
        cp2 = pltpu.async_copy(p2_hbm.at[idx_v.at[c0 + 2]], buf0, sem0)
        cp1.wait()
        pltpu.sync_copy(buf1, out_hbm.at[pl.ds(base + (c0 + 1) * CH, CH)])
        cp2.wait()
        return carry

    lax.fori_loop(0, NPAIR, body, 0)
    pltpu.sync_copy(buf0, out_hbm.at[pl.ds(base + (NCH - 1) * CH, CH)])


def kernel(coords, pe):
    flat = coords.reshape(B_TOTAL, 2)
    xs = flat[:, 0].reshape(GRID, SUB, LANE)
    ys = flat[:, 1].reshape(GRID, SUB, LANE)
    idx, p2 = _prep(xs, ys, pe)
    out = _sc_gather(
        p2.reshape(MAX_LEN * MAX_LEN, D_MODEL),
        idx.reshape(NW, NCH, CH),
    )
    return out.reshape(coords.shape[0], coords.shape[1], coords.shape[2], D_MODEL)


# speedup vs baseline: 1.1365x; 1.0034x over previous
import functools

import jax
import jax.numpy as jnp
from jax import lax
from jax.experimental import pallas as pl
from jax.experimental.pallas import tpu as pltpu
from jax.experimental.pallas import tpu_sc as plsc

D_MODEL = 256
MAX_LEN = 100
B_TOTAL = 16 * 128 * 50
GRID = 100
SUB = 8
LANE = 128
NC = 2
NS = 16
NW = NC * NS
BPW = B_TOTAL // NW
IDX_MINOR = 128
IDX_ROWS = BPW // IDX_MINOR
CH = 32
NCH = BPW // CH
NB = 10


def _prep_body(xs_ref, ys_ref, pe_row_ref, pe_full_ref, idx_ref, p2_ref):
    qx = (((xs_ref[0] + 50.0) / 100.0) * (MAX_LEN - 1)).astype(jnp.int32)
    qy = (((ys_ref[0] + 50.0) / 100.0) * (MAX_LEN - 1)).astype(jnp.int32)
    qx = jnp.clip(qx, 0, MAX_LEN - 1)
    qy = jnp.clip(qy, 0, MAX_LEN - 1)
    idx_ref[0] = qx * MAX_LEN + qy
    p2_ref[0] = pe_row_ref[0] + pe_full_ref[...]


def _prep(xs, ys, pe):
    return pl.pallas_call(
        _prep_body,
        grid=(GRID,),
        in_specs=[
            pl.BlockSpec((1, SUB, LANE), lambda i: (i, 0, 0)),
            pl.BlockSpec((1, SUB, LANE), lambda i: (i, 0, 0)),
            pl.BlockSpec((1, 1, D_MODEL), lambda i: (i, 0, 0)),
            pl.BlockSpec((MAX_LEN, D_MODEL), lambda i: (0, 0)),
        ],
        out_specs=[
            pl.BlockSpec((1, SUB, LANE), lambda i: (i, 0, 0)),
            pl.BlockSpec((1, MAX_LEN, D_MODEL), lambda i: (i, 0, 0)),
        ],
        out_shape=[
            jax.ShapeDtypeStruct((GRID, SUB, LANE), jnp.int32),
            jax.ShapeDtypeStruct((MAX_LEN, MAX_LEN, D_MODEL), jnp.float32),
        ],
    )(xs, ys, pe.reshape(MAX_LEN, 1, D_MODEL), pe)


@functools.partial(
    pl.kernel,
    mesh=plsc.VectorSubcoreMesh(core_axis_name="c", subcore_axis_name="s"),
    out_type=jax.ShapeDtypeStruct((B_TOTAL, D_MODEL), jnp.float32),
    scratch_types=(
        [pltpu.VMEM((IDX_ROWS, IDX_MINOR), jnp.int32)]
        + [pltpu.VMEM((CH, D_MODEL), jnp.float32) for _ in range(NB)]
        + [pltpu.SemaphoreType.DMA for _ in range(NB)]
    ),
)
def _sc_gather(p2_hbm, idx_hbm, out_hbm, idx_v, *bufs_sems):
    bufs = bufs_sems[:NB]
    sems = bufs_sems[NB:]
    wid = lax.axis_index("s") * NC + lax.axis_index("c")
    base = wid * BPW
    pltpu.sync_copy(idx_hbm.at[wid], idx_v)

    def idx_slice(c):
        return idx_v.at[c // 4, pl.ds((c % 4) * CH, CH)]

    for b in range(NB):
        pltpu.async_copy(p2_hbm.at[idx_slice(b)], bufs[b], sems[b])

    def body(g, carry):
        for b in range(NB):
            c = g * NB + b
            pltpu.make_async_copy(p2_hbm.at[idx_slice(c)], bufs[b], sems[b]).wait()
            pltpu.sync_copy(bufs[b], out_hbm.at[pl.ds(base + c * CH, CH)])
            nxt = c + NB

            @pl.when(nxt < NCH)
            def _():
                pltpu.async_copy(p2_hbm.at[idx_slice(nxt)], bufs[b], sems[b])

        return carry

    lax.fori_loop(0, NCH // NB, body, 0)


def kernel(coords, pe):
    flat = coords.reshape(B_TOTAL, 2)
    xs = flat[:, 0].reshape(GRID, SUB, LANE)
    ys = flat[:, 1].reshape(GRID, SUB, LANE)
    idx, p2 = _prep(xs, ys, pe)
    out = _sc_gather(
        p2.reshape(MAX_LEN * MAX_LEN, D_MODEL),
        idx.reshape(NW, IDX_ROWS, IDX_MINOR),
    )
    return out.reshape(coords.shape[0], coords.shape[1], coords.shape[2], D_MODEL)
